# baseline (speedup 1.0000x reference)
import functools

import jax
import jax.numpy as jnp
from jax import lax
from jax.experimental import pallas as pl
from jax.experimental.pallas import tpu as pltpu
from jax.experimental.pallas import tpu_sc as plsc

S = 2048
H = 1024
E = 8
K = 2
VD = 1536
CAP = 308
CPAD = 320
NCH = E * CPAD
LANES = 128
NW = 32
INVALID = CPAD - 1


def _route_body(x_ref, g_ref, idx_ref, w_ref, xp_ref):
    X = x_ref[...]
    ar = X[:, :H // 2].astype(jnp.bfloat16).astype(jnp.float32)
    br = X[:, H // 2:].astype(jnp.bfloat16).astype(jnp.float32)
    ai = lax.bitcast_convert_type(ar, jnp.int32)
    bi = lax.bitcast_convert_type(br, jnp.int32)
    xp_ref[...] = lax.shift_right_logical(ai, 16) | (bi & jnp.int32(-65536))
    G = g_ref[...]
    logits = jnp.dot(X, G, preferred_element_type=jnp.float32)
    lane = lax.broadcasted_iota(jnp.int32, (S, LANES), 1)
    emask = lane < E
    l = jnp.where(emask, logits, -1e30)
    m = jnp.max(l, axis=1, keepdims=True)
    p = jnp.where(emask, jnp.exp(l - m), 0.0)
    probs = p / jnp.sum(p, axis=1, keepdims=True)
    v1 = jnp.max(probs, axis=1, keepdims=True)
    a1 = jnp.min(jnp.where(probs == v1, lane, LANES), axis=1, keepdims=True)
    probs2 = jnp.where(lane == a1, -1.0, probs)
    v2 = jnp.max(probs2, axis=1, keepdims=True)
    a2 = jnp.min(jnp.where(probs2 == v2, lane, LANES), axis=1, keepdims=True)
    denom = v1 + v2
    w1 = v1 / denom
    w2 = v2 / denom
    c = ((lane == a1) | (lane == a2)).astype(jnp.float32)
    tri = (lax.broadcasted_iota(jnp.int32, (LANES, LANES), 0)
           > lax.broadcasted_iota(jnp.int32, (LANES, LANES), 1)).astype(jnp.float32)
    blocks = []
    off = jnp.zeros((1, LANES), jnp.float32)
    nb = S // LANES
    for bi in range(nb):
        cb = c[bi * LANES:(bi + 1) * LANES, :]
        blocks.append(jnp.dot(tri, cb, preferred_element_type=jnp.float32) + off)
        off = off + jnp.sum(cb, axis=0, keepdims=True)
    prefix = jnp.concatenate(blocks, axis=0)
    bet = jnp.broadcast_to(off, (S, LANES))
    m1 = (lane == a1)
    m2 = (lane == a2)
    rank1 = jnp.sum(jnp.where(m1, prefix, 0.0), axis=1, keepdims=True)
    rank2 = jnp.sum(jnp.where(m2, prefix, 0.0), axis=1, keepdims=True)
    bet1 = jnp.sum(jnp.where(m1, bet, 0.0), axis=1, keepdims=True)
    bet2 = jnp.sum(jnp.where(m2, bet, 0.0), axis=1, keepdims=True)
    p1 = (rank1 - bet1).astype(jnp.int32) + CAP
    p2 = (rank2 - bet2).astype(jnp.int32) + CAP
    ok1 = p1 >= 0
    ok2 = p2 >= 0
    idx1 = jnp.where(ok1, a1 * CPAD + p1, INVALID)
    idx2 = jnp.where(ok2, a2 * CPAD + p2, INVALID)
    we1 = jnp.where(ok1, w1, 0.0)
    we2 = jnp.where(ok2, w2, 0.0)
    idx_ref[...] = jnp.where(lane == 0, idx1, jnp.where(lane == 1, idx2, 0))
    w_ref[...] = jnp.where(lane == 0, we1, jnp.where(lane == 1, we2, 0.0))


def _route(xs, gWt):
    return pl.pallas_call(
        _route_body,
        out_shape=(jax.ShapeDtypeStruct((S, LANES), jnp.int32),
                   jax.ShapeDtypeStruct((S, LANES), jnp.float32),
                   jax.ShapeDtypeStruct((S, H // 2), jnp.int32)),
    )(xs, gWt)


def _wid():
    return lax.axis_index("s") * 2 + lax.axis_index("c")


def _dispatch_body(x_hbm, slots_hbm, chunks_hbm, slots_v, myidx_v, rows_v, sem):
    wid = _wid()
    per = NCH // NW
    base = wid * per
    zero = jnp.zeros((16,), jnp.int32)

    def zbody(i, _):
        myidx_v[pl.ds(i * 16, 16)] = zero
        return 0

    lax.fori_loop(0, per // 16, zbody, 0)
    pltpu.sync_copy(slots_hbm, slots_v)

    def sbody(i, _):
        iv = slots_v[pl.ds(i * 16, 16)]
        tok = (lax.iota(jnp.int32, 16) + i * 16) >> 1
        rel = iv - base
        plsc.store_scatter(myidx_v, [rel], tok,
                           mask=(rel >= 0) & (rel < per) & (iv != INVALID))
        return 0

    lax.fori_loop(0, (S * K) // 16, sbody, 0)
    copies = [pltpu.async_copy(
        x_hbm.at[myidx_v.at[pl.ds(c * GCH, GCH)]],
        rows_v.at[pl.ds(c * GCH, GCH)], sem) for c in range(per // GCH)]
    for cp in copies:
        cp.wait()
    pltpu.sync_copy(rows_v, chunks_hbm.at[pl.ds(wid * per, per)])


def _experts_body(ch_ref, wv_ref, wo_ref, idx_ref, w_ref, o_ref, z_acc):
    e = pl.program_id(0)
    ch32 = ch_ref[...]
    A = lax.bitcast_convert_type(ch32 << 16, jnp.float32).astype(jnp.bfloat16)
    B = lax.bitcast_convert_type(ch32 & jnp.int32(-65536),
                                 jnp.float32).astype(jnp.bfloat16)
    wv = wv_ref[0].astype(jnp.bfloat16)
    h = (jnp.dot(A, wv[:H // 2], preferred_element_type=jnp.float32)
         + jnp.dot(B, wv[H // 2:], preferred_element_type=jnp.float32))
    y = h * jax.nn.sigmoid(h)
    z = jnp.dot(y.astype(jnp.bfloat16),
                wo_ref[...].astype(jnp.bfloat16),
                preferred_element_type=jnp.float32)
    z_acc[pl.ds(e * CPAD, CPAD), :] = z.astype(jnp.bfloat16)

    @pl.when(e == E - 1)
    def _comb():
        lane = lax.broadcasted_iota(jnp.int32, (S, NCH), 1)
        P = (jnp.where(lane == idx_ref[:, 0:1], w_ref[:, 0:1], 0.0)
             + jnp.where(lane == idx_ref[:, 1:2], w_ref[:, 1:2], 0.0))
        o_ref[...] = jnp.dot(P.astype(jnp.bfloat16), z_acc[...],
                             preferred_element_type=jnp.float32)


def _experts(chunks, W_v, W_o, idx_full, w_full):
    return pl.pallas_call(
        _experts_body,
        grid=(E,),
        in_specs=[pl.BlockSpec((CPAD, H // 2), lambda e: (e, 0)),
                  pl.BlockSpec((1, H, VD), lambda e: (e, 0, 0)),
                  pl.BlockSpec((VD, H), lambda e: (0, 0)),
                  pl.BlockSpec((S, LANES), lambda e: (0, 0)),
                  pl.BlockSpec((S, LANES), lambda e: (0, 0))],
        out_specs=pl.BlockSpec((S, H), lambda e: (0, 0)),
        out_shape=jax.ShapeDtypeStruct((S, H), jnp.float32),
        scratch_shapes=[pltpu.VMEM((NCH, H), jnp.bfloat16)],
    )(chunks, W_v, W_o, idx_full, w_full)


GCH = 8


@functools.lru_cache(maxsize=None)
def _sc_kernels():
    mesh = plsc.VectorSubcoreMesh(core_axis_name="c", subcore_axis_name="s")
    sc_params = pltpu.CompilerParams(needs_layout_passes=False)
    dispatch = pl.kernel(
        _dispatch_body,
        out_type=jax.ShapeDtypeStruct((NCH, H // 2), jnp.int32),
        mesh=mesh,
        compiler_params=sc_params,
        scratch_types=[
            pltpu.VMEM((S * K,), jnp.int32),
            pltpu.VMEM((NCH // NW,), jnp.int32),
            pltpu.VMEM((NCH // NW, H // 2), jnp.int32),
            pltpu.SemaphoreType.DMA,
        ],
    )
    return dispatch


def kernel(x, gate_W, W_v, W_o):
    b, s0, d = x.shape
    xs = x.reshape(s0, d)
    gWt = jnp.zeros((d, LANES), x.dtype).at[:, :E].set(gate_W.T)
    idx_full, w_full, xb = _route(xs, gWt)
    idx2 = idx_full[:, :K]
    slots = idx2.reshape(-1)
    dispatch = _sc_kernels()
    chunks = dispatch(xb, slots)
    out = _experts(chunks, W_v, W_o, idx_full, w_full)
    return out.reshape(b, s0, d)


# speedup vs baseline: 1.0277x; 1.0277x over previous
import functools

import jax
import jax.numpy as jnp
from jax import lax
from jax.experimental import pallas as pl
from jax.experimental.pallas import tpu as pltpu
from jax.experimental.pallas import tpu_sc as plsc

S = 2048
H = 1024
E = 8
K = 2
VD = 1536
CAP = 308
CPAD = 320
NCH = E * CPAD
LANES = 128
NW = 32
INVALID = CPAD - 1


def _route_body(x_ref, g_ref, idx_ref, w_ref, xp_ref):
    X = x_ref[...]
    ar = X[:, :H // 2].astype(jnp.bfloat16).astype(jnp.float32)
    br = X[:, H // 2:].astype(jnp.bfloat16).astype(jnp.float32)
    ai = lax.bitcast_convert_type(ar, jnp.int32)
    bi = lax.bitcast_convert_type(br, jnp.int32)
    xp_ref[...] = lax.shift_right_logical(ai, 16) | (bi & jnp.int32(-65536))
    G = g_ref[...]
    logits = lax.dot_general(X, G, (((1,), (1,)), ((), ())),
                             preferred_element_type=jnp.float32)
    lane = lax.broadcasted_iota(jnp.int32, (S, E), 1)
    m = jnp.max(logits, axis=1, keepdims=True)
    p = jnp.exp(logits - m)
    probs = p / jnp.sum(p, axis=1, keepdims=True)
    v1 = jnp.max(probs, axis=1, keepdims=True)
    a1 = jnp.min(jnp.where(probs == v1, lane, E), axis=1, keepdims=True)
    probs2 = jnp.where(lane == a1, -1.0, probs)
    v2 = jnp.max(probs2, axis=1, keepdims=True)
    a2 = jnp.min(jnp.where(probs2 == v2, lane, LANES), axis=1, keepdims=True)
    denom = v1 + v2
    w1 = v1 / denom
    w2 = v2 / denom
    c = ((lane == a1) | (lane == a2)).astype(jnp.float32)
    tri = (lax.broadcasted_iota(jnp.int32, (LANES, LANES), 0)
           > lax.broadcasted_iota(jnp.int32, (LANES, LANES), 1)).astype(jnp.float32)
    blocks = []
    off = jnp.zeros((1, E), jnp.float32)
    nb = S // LANES
    for bi in range(nb):
        cb = c[bi * LANES:(bi + 1) * LANES, :]
        blocks.append(jnp.dot(tri, cb, preferred_element_type=jnp.float32) + off)
        off = off + jnp.sum(cb, axis=0, keepdims=True)
    prefix = jnp.concatenate(blocks, axis=0)
    bet = jnp.broadcast_to(off, (S, E))
    m1 = (lane == a1)
    m2 = (lane == a2)
    rank1 = jnp.sum(jnp.where(m1, prefix, 0.0), axis=1, keepdims=True)
    rank2 = jnp.sum(jnp.where(m2, prefix, 0.0), axis=1, keepdims=True)
    bet1 = jnp.sum(jnp.where(m1, bet, 0.0), axis=1, keepdims=True)
    bet2 = jnp.sum(jnp.where(m2, bet, 0.0), axis=1, keepdims=True)
    p1 = (rank1 - bet1).astype(jnp.int32) + CAP
    p2 = (rank2 - bet2).astype(jnp.int32) + CAP
    ok1 = p1 >= 0
    ok2 = p2 >= 0
    idx1 = jnp.where(ok1, a1 * CPAD + p1, INVALID)
    idx2 = jnp.where(ok2, a2 * CPAD + p2, INVALID)
    we1 = jnp.where(ok1, w1, 0.0)
    we2 = jnp.where(ok2, w2, 0.0)
    olane = lax.broadcasted_iota(jnp.int32, (S, LANES), 1)
    idx_ref[...] = jnp.where(olane == 0, idx1, jnp.where(olane == 1, idx2, 0))
    w_ref[...] = jnp.where(olane == 0, we1, jnp.where(olane == 1, we2, 0.0))


def _route(xs, gWt):
    return pl.pallas_call(
        _route_body,
        out_shape=(jax.ShapeDtypeStruct((S, LANES), jnp.int32),
                   jax.ShapeDtypeStruct((S, LANES), jnp.float32),
                   jax.ShapeDtypeStruct((S, H // 2), jnp.int32)),
    )(xs, gWt)


def _wid():
    return lax.axis_index("s") * 2 + lax.axis_index("c")


def _dispatch_body(x_hbm, slots_hbm, chunks_hbm, slots_v, myidx_v, rows_v, sem):
    wid = _wid()
    per = NCH // NW
    base = wid * per
    zero = jnp.zeros((16,), jnp.int32)

    def zbody(i, _):
        myidx_v[pl.ds(i * 16, 16)] = zero
        return 0

    lax.fori_loop(0, per // 16, zbody, 0)
    pltpu.sync_copy(slots_hbm, slots_v)

    def sbody(i, _):
        iv = slots_v[pl.ds(i * 16, 16)]
        tok = (lax.iota(jnp.int32, 16) + i * 16) >> 1
        rel = iv - base
        plsc.store_scatter(myidx_v, [rel], tok,
                           mask=(rel >= 0) & (rel < per) & (iv != INVALID))
        return 0

    lax.fori_loop(0, (S * K) // 16, sbody, 0)
    copies = [pltpu.async_copy(
        x_hbm.at[myidx_v.at[pl.ds(c * GCH, GCH)]],
        rows_v.at[pl.ds(c * GCH, GCH)], sem) for c in range(per // GCH)]
    for cp in copies:
        cp.wait()
    pltpu.sync_copy(rows_v, chunks_hbm.at[pl.ds(wid * per, per)])


def _experts_body(ch_ref, wv_ref, wo_ref, idx_ref, w_ref, o_ref, z_acc):
    e = pl.program_id(0)
    ch32 = ch_ref[...]
    A = lax.bitcast_convert_type(ch32 << 16, jnp.float32).astype(jnp.bfloat16)
    B = lax.bitcast_convert_type(ch32 & jnp.int32(-65536),
                                 jnp.float32).astype(jnp.bfloat16)
    wv = wv_ref[0].astype(jnp.bfloat16)
    h = (jnp.dot(A, wv[:H // 2], preferred_element_type=jnp.float32)
         + jnp.dot(B, wv[H // 2:], preferred_element_type=jnp.float32))
    y = h * jax.nn.sigmoid(h)
    z = jnp.dot(y.astype(jnp.bfloat16),
                wo_ref[...].astype(jnp.bfloat16),
                preferred_element_type=jnp.float32)
    z_acc[pl.ds(e * CPAD, CPAD), :] = z.astype(jnp.bfloat16)

    @pl.when(e == E - 1)
    def _comb():
        lane = lax.broadcasted_iota(jnp.int32, (S, NCH), 1)
        P = (jnp.where(lane == idx_ref[:, 0:1], w_ref[:, 0:1], 0.0)
             + jnp.where(lane == idx_ref[:, 1:2], w_ref[:, 1:2], 0.0))
        o_ref[...] = jnp.dot(P.astype(jnp.bfloat16), z_acc[...],
                             preferred_element_type=jnp.float32)


def _experts(chunks, W_v, W_o, idx_full, w_full):
    return pl.pallas_call(
        _experts_body,
        grid=(E,),
        in_specs=[pl.BlockSpec((CPAD, H // 2), lambda e: (e, 0)),
                  pl.BlockSpec((1, H, VD), lambda e: (e, 0, 0)),
                  pl.BlockSpec((VD, H), lambda e: (0, 0)),
                  pl.BlockSpec((S, LANES), lambda e: (0, 0)),
                  pl.BlockSpec((S, LANES), lambda e: (0, 0))],
        out_specs=pl.BlockSpec((S, H), lambda e: (0, 0)),
        out_shape=jax.ShapeDtypeStruct((S, H), jnp.float32),
        scratch_shapes=[pltpu.VMEM((NCH, H), jnp.bfloat16)],
    )(chunks, W_v, W_o, idx_full, w_full)


GCH = 8


@functools.lru_cache(maxsize=None)
def _sc_kernels():
    mesh = plsc.VectorSubcoreMesh(core_axis_name="c", subcore_axis_name="s")
    sc_params = pltpu.CompilerParams(needs_layout_passes=False)
    dispatch = pl.kernel(
        _dispatch_body,
        out_type=jax.ShapeDtypeStruct((NCH, H // 2), jnp.int32),
        mesh=mesh,
        compiler_params=sc_params,
        scratch_types=[
            pltpu.VMEM((S * K,), jnp.int32),
            pltpu.VMEM((NCH // NW,), jnp.int32),
            pltpu.VMEM((NCH // NW, H // 2), jnp.int32),
            pltpu.SemaphoreType.DMA,
        ],
    )
    return dispatch


def kernel(x, gate_W, W_v, W_o):
    b, s0, d = x.shape
    xs = x.reshape(s0, d)
    idx_full, w_full, xb = _route(xs, gate_W)
    slots = idx_full[:, :K].reshape(-1)
    dispatch = _sc_kernels()
    chunks = dispatch(xb, slots)
    out = _experts(chunks, W_v, W_o, idx_full, w_full)
    return out.reshape(b, s0, d)
